# baseline (speedup 1.0000x reference)
import functools

import jax
import jax.numpy as jnp
from jax import lax
from jax.experimental import pallas as pl
from jax.experimental.pallas import tpu as pltpu
from jax.experimental.pallas import tpu_sc as plsc

N_DAYS = 366
D_MODEL = 256
BATCH = 16384

NC = 2
NS = 16
NW = NC * NS
B_PER_W = BATCH // NW
CHUNK = 64
N_CHUNK = B_PER_W // CHUNK
NBUF = 6
NREP = 8

_mesh = plsc.VectorSubcoreMesh(core_axis_name="c", subcore_axis_name="s")


@functools.partial(
    pl.kernel,
    mesh=_mesh,
    out_type=jax.ShapeDtypeStruct((BATCH, D_MODEL), jnp.float32),
    scratch_types=[
        pltpu.VMEM((N_CHUNK, CHUNK), jnp.int32),
        *[pltpu.VMEM((CHUNK, D_MODEL), jnp.float32) for _ in range(NBUF)],
        pltpu.SemaphoreType.DMA,
        *[pltpu.SemaphoreType.DMA for _ in range(2 * NBUF)],
    ],
)
def _gather_kernel(x_hbm, repl_hbm, out_hbm, idx_v, *scratch):
    bufs = scratch[:NBUF]
    isem = scratch[NBUF]
    gsems = scratch[NBUF + 1:NBUF + 1 + NBUF]
    ssems = scratch[NBUF + 1 + NBUF:]
    sid = lax.axis_index("s")
    wid = sid * NC + lax.axis_index("c")
    base = wid * B_PER_W

    pltpu.sync_copy(x_hbm.at[pl.ds(wid * N_CHUNK, N_CHUNK)], idx_v)

    roff = (wid % NREP) * N_DAYS

    def fixup(j):
        for k in range(CHUNK // 16):
            v = idx_v[j, pl.ds(k * 16, 16)]
            idx_v[j, pl.ds(k * 16, 16)] = jnp.where(v == 0, N_DAYS - 1, v - 1) + roff

    def gather(j):
        return pltpu.async_copy(repl_hbm.at[idx_v.at[j]], bufs[j % NBUF], gsems[j % NBUF])

    def scatter(j):
        return pltpu.async_copy(
            bufs[j % NBUF], out_hbm.at[pl.ds(base + j * CHUNK, CHUNK)], ssems[j % NBUF]
        )

    gcp = [None] * N_CHUNK
    scp = [None] * N_CHUNK
    for j in range(N_CHUNK):
        fixup(j)
        if j < NBUF:
            gcp[j] = gather(j)
    for j in range(N_CHUNK):
        gcp[j].wait()
        scp[j] = scatter(j)
        if j + NBUF < N_CHUNK:
            scp[j].wait()
            gcp[j + NBUF] = gather(j + NBUF)
    for j in range(max(0, N_CHUNK - NBUF), N_CHUNK):
        scp[j].wait()


def kernel(x, pe):
    repl = jnp.tile(pe, (NREP, 1))
    x2d = x.astype(jnp.int32).reshape(BATCH // CHUNK, CHUNK)
    return _gather_kernel(x2d, repl)


# speedup vs baseline: 1.0201x; 1.0201x over previous
import functools

import jax
import jax.numpy as jnp
from jax import lax
from jax.experimental import pallas as pl
from jax.experimental.pallas import tpu as pltpu
from jax.experimental.pallas import tpu_sc as plsc

N_DAYS = 366
D_MODEL = 256
BATCH = 16384

NC = 2
NS = 16
NW = NC * NS
B_PER_W = BATCH // NW
CHUNK = 128
N_CHUNK = B_PER_W // CHUNK
NBUF = 3
NREP = 8
N_DAYS_PAD = 368

_mesh = plsc.VectorSubcoreMesh(core_axis_name="c", subcore_axis_name="s")


@functools.partial(
    pl.kernel,
    mesh=_mesh,
    out_type=jax.ShapeDtypeStruct((BATCH, D_MODEL), jnp.float32),
    scratch_types=[
        pltpu.VMEM((N_CHUNK, CHUNK), jnp.int32),
        *[pltpu.VMEM((CHUNK, D_MODEL), jnp.float32) for _ in range(NBUF)],
        pltpu.SemaphoreType.DMA,
        *[pltpu.SemaphoreType.DMA for _ in range(2 * NBUF)],
    ],
)
def _gather_kernel(x_hbm, repl_hbm, out_hbm, idx_v, *scratch):
    bufs = scratch[:NBUF]
    isem = scratch[NBUF]
    gsems = scratch[NBUF + 1:NBUF + 1 + NBUF]
    ssems = scratch[NBUF + 1 + NBUF:]
    sid = lax.axis_index("s")
    wid = sid * NC + lax.axis_index("c")
    base = wid * B_PER_W

    pltpu.sync_copy(x_hbm.at[pl.ds(wid * N_CHUNK, N_CHUNK)], idx_v)

    roff = (wid % NREP) * N_DAYS_PAD

    def fixup(j):
        for k in range(CHUNK // 16):
            v = idx_v[j, pl.ds(k * 16, 16)]
            idx_v[j, pl.ds(k * 16, 16)] = jnp.where(v == 0, N_DAYS - 1, v - 1) + roff

    def gather(j):
        return pltpu.async_copy(repl_hbm.at[idx_v.at[j]], bufs[j % NBUF], gsems[j % NBUF])

    def scatter(j):
        return pltpu.async_copy(
            bufs[j % NBUF], out_hbm.at[pl.ds(base + j * CHUNK, CHUNK)], ssems[j % NBUF]
        )

    gcp = [None] * N_CHUNK
    scp = [None] * N_CHUNK
    for j in range(N_CHUNK):
        fixup(j)
        if j < NBUF:
            gcp[j] = gather(j)
    for j in range(N_CHUNK):
        gcp[j].wait()
        scp[j] = scatter(j)
        if j + NBUF < N_CHUNK:
            scp[j].wait()
            gcp[j + NBUF] = gather(j + NBUF)
    for j in range(max(0, N_CHUNK - NBUF), N_CHUNK):
        scp[j].wait()


def kernel(x, pe):
    repl = jnp.tile(jnp.pad(pe, ((0, N_DAYS_PAD - N_DAYS), (0, 0))), (NREP, 1))
    x2d = x.astype(jnp.int32).reshape(BATCH // CHUNK, CHUNK)
    return _gather_kernel(x2d, repl)
